# baseline (speedup 1.0000x reference)
import functools

import jax
import jax.numpy as jnp
from jax.experimental import pallas as pl
from jax.experimental.pallas import tpu as pltpu

_NEG_SLOPE = 0.01
_BN_EPS = 1e-5
_D_TILE = 8
_LEAD = 128


def _conv_tile(xm_ref, xh_ref, wk_ref, wbuf_ref, col_ref, *, c_in, hw):
    ts = _D_TILE * hw
    tspan = ts + 2 * hw

    hi = _LEAD + hw + ts
    wbuf_ref[:, 0:_LEAD] = jnp.zeros((c_in, _LEAD), jnp.float32)
    wbuf_ref[:, _LEAD:_LEAD + hw] = xh_ref[:, 0, :]
    wbuf_ref[:, _LEAD + hw:hi] = xm_ref[...]
    wbuf_ref[:, hi:hi + hw] = xh_ref[:, 1, :]
    wbuf_ref[:, hi + hw:hi + hw + _LEAD] = jnp.zeros((c_in, _LEAD), jnp.float32)

    vw = wbuf_ref[...]
    wi = jax.lax.broadcasted_iota(jnp.int32, (c_in, tspan), 1)
    w_pos = wi % 32
    h_pos = (wi // 32) % 32
    for b in range(3):
        for cw in range(3):
            t = b * 3 + cw
            s = _LEAD + (b - 1) * 32 + (cw - 1)
            v = vw[:, s:s + tspan]
            bad = None
            if b == 0:
                bad = h_pos == 0
            elif b == 2:
                bad = h_pos == 31
            if cw == 0:
                bad = (w_pos == 0) if bad is None else bad | (w_pos == 0)
            elif cw == 2:
                bad = (w_pos == 31) if bad is None else bad | (w_pos == 31)
            if bad is not None:
                v = jnp.where(bad, 0.0, v)
            col_ref[t * c_in:(t + 1) * c_in, :] = v.astype(jnp.bfloat16)

    acc = None
    for a in range(3):
        xs = col_ref[:, a * hw:a * hw + ts]
        p = jax.lax.dot_general(
            xs, wk_ref[a],
            dimension_numbers=(((0,), (0,)), ((), ())),
            preferred_element_type=jnp.float32)
        acc = p if acc is None else acc + p
    return acc.T


def _stats_kernel(xm_ref, xh_ref, wk_ref, stats_ref, wbuf_ref, col_ref, **kw):
    yt = _conv_tile(xm_ref, xh_ref, wk_ref, wbuf_ref, col_ref, **kw)
    stats_ref[:, 0:1] = jnp.sum(yt, axis=-1, keepdims=True)
    stats_ref[:, 1:2] = jnp.sum(yt * yt, axis=-1, keepdims=True)


def _apply_kernel(xm_ref, xh_ref, wk_ref, scale_ref, shift_ref, o_ref,
                  wbuf_ref, col_ref, **kw):
    yt = _conv_tile(xm_ref, xh_ref, wk_ref, wbuf_ref, col_ref, **kw)
    z = yt * scale_ref[...] + shift_ref[...]
    o_ref[...] = jnp.where(z >= 0.0, z, _NEG_SLOPE * z)


def kernel(w, gamma, beta, seqs, seqL):
    del seqL
    N, C, D, H, W = seqs.shape
    Cout = w.shape[0]
    hw = H * W
    ms = D * hw
    n_tiles = D // _D_TILE
    ts = _D_TILE * hw
    wspan = 2 * _LEAD + 2 * hw + ts

    xb = seqs.reshape(N, C, ms)
    x4 = seqs.reshape(N, C, D, hw)
    z1 = jnp.zeros((N, C, 1, hw), seqs.dtype)
    lo = jnp.concatenate(
        [z1, x4[:, :, _D_TILE - 1::_D_TILE][:, :, :n_tiles - 1]], axis=2)
    hi = jnp.concatenate([x4[:, :, _D_TILE::_D_TILE], z1], axis=2)
    xh = jnp.stack([lo, hi], axis=3)

    wk = w.transpose(2, 3, 4, 1, 0).reshape(3, 9 * C, Cout).astype(jnp.bfloat16)

    kw_args = dict(c_in=C, hw=hw)
    grid = (N, n_tiles)
    xm_spec = pl.BlockSpec((None, C, ts), lambda n, i: (n, 0, i))
    xh_spec = pl.BlockSpec((None, C, None, 2, hw), lambda n, i: (n, 0, i, 0, 0))
    wk_spec = pl.BlockSpec((3, 9 * C, Cout), lambda n, i: (0, 0, 0))
    scratch = [pltpu.VMEM((C, wspan), jnp.float32),
               pltpu.VMEM((9 * C, ts + 2 * hw), jnp.bfloat16)]

    cost1 = pl.CostEstimate(
        flops=2 * N * ms * (27 * C) * Cout,
        transcendentals=0,
        bytes_accessed=N * C * (ms + 2 * n_tiles * hw) * 4
                       + 3 * 9 * C * Cout * 2 + N * n_tiles * Cout * 2 * 4)

    part_stats = pl.pallas_call(
        functools.partial(_stats_kernel, **kw_args),
        out_shape=jax.ShapeDtypeStruct((N, n_tiles, Cout, 2), jnp.float32),
        grid=grid,
        in_specs=[xm_spec, xh_spec, wk_spec],
        out_specs=pl.BlockSpec((None, None, Cout, 2), lambda n, i: (n, i, 0, 0)),
        scratch_shapes=scratch,
        compiler_params=pltpu.CompilerParams(
            dimension_semantics=("parallel", "parallel")),
        cost_estimate=cost1,
    )(xb, xh, wk)

    M = N * ms
    stats = jnp.sum(part_stats, axis=(0, 1))
    mean = stats[:, 0] / M
    var = jnp.maximum(stats[:, 1] / M - mean * mean, 0.0)
    scale = gamma / jnp.sqrt(var + _BN_EPS)
    shift = beta - mean * scale
    scale_c = scale.reshape(Cout, 1).astype(jnp.float32)
    shift_c = shift.reshape(Cout, 1).astype(jnp.float32)

    cost2 = pl.CostEstimate(
        flops=2 * N * ms * (27 * C) * Cout,
        transcendentals=0,
        bytes_accessed=N * C * (ms + 2 * n_tiles * hw) * 4
                       + N * Cout * ms * 4 + 2 * Cout * 4)

    out_cm = pl.pallas_call(
        functools.partial(_apply_kernel, **kw_args),
        out_shape=jax.ShapeDtypeStruct((N, Cout, ms), jnp.float32),
        grid=grid,
        in_specs=[xm_spec, xh_spec, wk_spec,
                  pl.BlockSpec((Cout, 1), lambda n, i: (0, 0)),
                  pl.BlockSpec((Cout, 1), lambda n, i: (0, 0))],
        out_specs=pl.BlockSpec((None, Cout, ts), lambda n, i: (n, 0, i)),
        scratch_shapes=scratch,
        compiler_params=pltpu.CompilerParams(
            dimension_semantics=("parallel", "parallel")),
        cost_estimate=cost2,
    )(xb, xh, wk, scale_c, shift_c)

    return out_cm.reshape(N, Cout, D, H, W)


# speedup vs baseline: 1.7854x; 1.7854x over previous
import functools

import jax
import jax.numpy as jnp
from jax.experimental import pallas as pl
from jax.experimental.pallas import tpu as pltpu

_NEG_SLOPE = 0.01
_BN_EPS = 1e-5
_D_TILE = 8
_LEAD = 128


def _conv_tile(xb_ref, wk_ref, wbuf_ref, col_ref, *, c_in, hw, n_tiles):
    mt = pl.program_id(1)
    ts = _D_TILE * hw
    tspan = ts + 2 * hw
    base = mt * ts

    lo = _LEAD
    hi = _LEAD + hw + ts
    wbuf_ref[:, 0:_LEAD] = jnp.zeros((c_in, _LEAD), jnp.float32)
    wbuf_ref[:, hi + hw:hi + hw + _LEAD] = jnp.zeros((c_in, _LEAD), jnp.float32)

    @pl.when(mt == 0)
    def _():
        wbuf_ref[:, lo:lo + hw] = jnp.zeros((c_in, hw), jnp.float32)

    @pl.when(mt > 0)
    def _():
        wbuf_ref[:, lo:lo + hw] = xb_ref[:, pl.ds(base - hw, hw)]

    wbuf_ref[:, lo + hw:lo + hw + ts] = xb_ref[:, pl.ds(base, ts)]

    @pl.when(mt == n_tiles - 1)
    def _():
        wbuf_ref[:, hi:hi + hw] = jnp.zeros((c_in, hw), jnp.float32)

    @pl.when(mt < n_tiles - 1)
    def _():
        wbuf_ref[:, hi:hi + hw] = xb_ref[:, pl.ds(base + ts, hw)]

    vw = wbuf_ref[...]
    wi = jax.lax.broadcasted_iota(jnp.int32, (c_in, tspan), 1)
    w_pos = wi % 32
    h_pos = (wi // 32) % 32
    for b in range(3):
        for cw in range(3):
            t = b * 3 + cw
            s = _LEAD + (b - 1) * 32 + (cw - 1)
            v = vw[:, s:s + tspan]
            bad = None
            if b == 0:
                bad = h_pos == 0
            elif b == 2:
                bad = h_pos == 31
            if cw == 0:
                bad = (w_pos == 0) if bad is None else bad | (w_pos == 0)
            elif cw == 2:
                bad = (w_pos == 31) if bad is None else bad | (w_pos == 31)
            if bad is not None:
                v = jnp.where(bad, 0.0, v)
            col_ref[t * c_in:(t + 1) * c_in, :] = v.astype(jnp.bfloat16)

    acc = None
    for a in range(3):
        xs = col_ref[:, a * hw:a * hw + ts]
        p = jax.lax.dot_general(
            xs, wk_ref[a],
            dimension_numbers=(((0,), (0,)), ((), ())),
            preferred_element_type=jnp.float32)
        acc = p if acc is None else acc + p
    return acc.T


def _conv_stats_kernel(xb_ref, wk_ref, y_ref, stats_ref, wbuf_ref, col_ref,
                       **kw):
    yt = _conv_tile(xb_ref, wk_ref, wbuf_ref, col_ref, **kw)
    y_ref[...] = yt.astype(y_ref.dtype)
    stats_ref[:, 0:1] = jnp.sum(yt, axis=-1, keepdims=True)
    stats_ref[:, 1:2] = jnp.sum(yt * yt, axis=-1, keepdims=True)


def _bn_act_kernel(y_ref, scale_ref, shift_ref, o_ref):
    y = y_ref[...].astype(jnp.float32) * scale_ref[...] + shift_ref[...]
    o_ref[...] = jnp.where(y >= 0.0, y, _NEG_SLOPE * y)


def kernel(w, gamma, beta, seqs, seqL):
    del seqL
    N, C, D, H, W = seqs.shape
    Cout = w.shape[0]
    hw = H * W
    ms = D * hw
    n_tiles = D // _D_TILE
    ts = _D_TILE * hw
    wspan = 2 * _LEAD + 2 * hw + ts

    xb = seqs.reshape(N, C, ms)
    wk = w.transpose(2, 3, 4, 1, 0).reshape(3, 9 * C, Cout).astype(jnp.bfloat16)

    kw_args = dict(c_in=C, hw=hw, n_tiles=n_tiles)
    grid = (N, n_tiles)
    xb_spec = pl.BlockSpec((None, C, ms), lambda n, i: (n, 0, 0))
    wk_spec = pl.BlockSpec((3, 9 * C, Cout), lambda n, i: (0, 0, 0))
    scratch = [pltpu.VMEM((C, wspan), jnp.float32),
               pltpu.VMEM((9 * C, ts + 2 * hw), jnp.bfloat16)]

    cost1 = pl.CostEstimate(
        flops=2 * N * ms * (27 * C) * Cout,
        transcendentals=0,
        bytes_accessed=N * C * ms * 4 + 3 * 9 * C * Cout * 2
                       + N * Cout * ms * 2 + N * n_tiles * Cout * 2 * 4)

    y_cm, part_stats = pl.pallas_call(
        functools.partial(_conv_stats_kernel, **kw_args),
        out_shape=(jax.ShapeDtypeStruct((N, Cout, ms), jnp.bfloat16),
                   jax.ShapeDtypeStruct((N, n_tiles, Cout, 2), jnp.float32)),
        grid=grid,
        in_specs=[xb_spec, wk_spec],
        out_specs=[pl.BlockSpec((None, Cout, ts), lambda n, i: (n, 0, i)),
                   pl.BlockSpec((None, None, Cout, 2), lambda n, i: (n, i, 0, 0))],
        scratch_shapes=scratch,
        compiler_params=pltpu.CompilerParams(
            dimension_semantics=("parallel", "arbitrary")),
        cost_estimate=cost1,
    )(xb, wk)

    M = N * ms
    stats = jnp.sum(part_stats, axis=(0, 1))
    mean = stats[:, 0] / M
    var = jnp.maximum(stats[:, 1] / M - mean * mean, 0.0)
    scale = gamma / jnp.sqrt(var + _BN_EPS)
    shift = beta - mean * scale
    scale_c = scale.reshape(Cout, 1).astype(jnp.float32)
    shift_c = shift.reshape(Cout, 1).astype(jnp.float32)

    cost2 = pl.CostEstimate(
        flops=4 * N * ms * Cout,
        transcendentals=0,
        bytes_accessed=N * ms * Cout * (2 + 4) + 2 * Cout * 4)

    out_cm = pl.pallas_call(
        _bn_act_kernel,
        out_shape=jax.ShapeDtypeStruct((N, Cout, ms), jnp.float32),
        grid=grid,
        in_specs=[pl.BlockSpec((None, Cout, ts), lambda n, i: (n, 0, i)),
                  pl.BlockSpec((Cout, 1), lambda n, i: (0, 0)),
                  pl.BlockSpec((Cout, 1), lambda n, i: (0, 0))],
        out_specs=pl.BlockSpec((None, Cout, ts), lambda n, i: (n, 0, i)),
        compiler_params=pltpu.CompilerParams(
            dimension_semantics=("parallel", "parallel")),
        cost_estimate=cost2,
    )(y_cm, scale_c, shift_c)

    return out_cm.reshape(N, Cout, D, H, W)


# speedup vs baseline: 1.8764x; 1.0510x over previous
import functools

import jax
import jax.numpy as jnp
from jax.experimental import pallas as pl
from jax.experimental.pallas import tpu as pltpu

_NEG_SLOPE = 0.01
_BN_EPS = 1e-5
_D_TILE = 16
_LEAD = 128


def _conv_tile(xb_ref, wk_ref, wbuf_ref, col_ref, *, c_in, hw, n_tiles):
    mt = pl.program_id(1)
    ts = _D_TILE * hw
    tspan = ts + 2 * hw
    base = mt * ts

    lo = _LEAD
    hi = _LEAD + hw + ts
    wbuf_ref[:, 0:_LEAD] = jnp.zeros((c_in, _LEAD), jnp.float32)
    wbuf_ref[:, hi + hw:hi + hw + _LEAD] = jnp.zeros((c_in, _LEAD), jnp.float32)

    @pl.when(mt == 0)
    def _():
        wbuf_ref[:, lo:lo + hw] = jnp.zeros((c_in, hw), jnp.float32)

    @pl.when(mt > 0)
    def _():
        wbuf_ref[:, lo:lo + hw] = xb_ref[:, pl.ds(base - hw, hw)]

    wbuf_ref[:, lo + hw:lo + hw + ts] = xb_ref[:, pl.ds(base, ts)]

    @pl.when(mt == n_tiles - 1)
    def _():
        wbuf_ref[:, hi:hi + hw] = jnp.zeros((c_in, hw), jnp.float32)

    @pl.when(mt < n_tiles - 1)
    def _():
        wbuf_ref[:, hi:hi + hw] = xb_ref[:, pl.ds(base + ts, hw)]

    vw = wbuf_ref[...]
    wi = jax.lax.broadcasted_iota(jnp.int32, (c_in, tspan), 1)
    w_pos = wi % 32
    h_pos = (wi // 32) % 32
    for b in range(3):
        for cw in range(3):
            t = b * 3 + cw
            s = _LEAD + (b - 1) * 32 + (cw - 1)
            v = vw[:, s:s + tspan]
            bad = None
            if b == 0:
                bad = h_pos == 0
            elif b == 2:
                bad = h_pos == 31
            if cw == 0:
                bad = (w_pos == 0) if bad is None else bad | (w_pos == 0)
            elif cw == 2:
                bad = (w_pos == 31) if bad is None else bad | (w_pos == 31)
            if bad is not None:
                v = jnp.where(bad, 0.0, v)
            col_ref[t * c_in:(t + 1) * c_in, :] = v.astype(jnp.bfloat16)

    acc = None
    for a in range(3):
        xs = col_ref[:, a * hw:a * hw + ts]
        p = jax.lax.dot_general(
            xs, wk_ref[a],
            dimension_numbers=(((0,), (0,)), ((), ())),
            preferred_element_type=jnp.float32)
        acc = p if acc is None else acc + p
    return acc.T


def _conv_stats_kernel(xb_ref, wk_ref, y_ref, stats_ref, wbuf_ref, col_ref,
                       **kw):
    yt = _conv_tile(xb_ref, wk_ref, wbuf_ref, col_ref, **kw)
    y_ref[...] = yt.astype(y_ref.dtype)
    stats_ref[:, 0:1] = jnp.sum(yt, axis=-1, keepdims=True)
    stats_ref[:, 1:2] = jnp.sum(yt * yt, axis=-1, keepdims=True)


def _bn_act_kernel(y_ref, scale_ref, shift_ref, o_ref):
    y = y_ref[...].astype(jnp.float32) * scale_ref[...] + shift_ref[...]
    o_ref[...] = jnp.where(y >= 0.0, y, _NEG_SLOPE * y)


def kernel(w, gamma, beta, seqs, seqL):
    del seqL
    N, C, D, H, W = seqs.shape
    Cout = w.shape[0]
    hw = H * W
    ms = D * hw
    n_tiles = D // _D_TILE
    ts = _D_TILE * hw
    wspan = 2 * _LEAD + 2 * hw + ts

    xb = seqs.reshape(N, C, ms)
    wk = w.transpose(2, 3, 4, 1, 0).reshape(3, 9 * C, Cout).astype(jnp.bfloat16)

    kw_args = dict(c_in=C, hw=hw, n_tiles=n_tiles)
    grid = (N, n_tiles)
    xb_spec = pl.BlockSpec((None, C, ms), lambda n, i: (n, 0, 0))
    wk_spec = pl.BlockSpec((3, 9 * C, Cout), lambda n, i: (0, 0, 0))
    scratch = [pltpu.VMEM((C, wspan), jnp.float32),
               pltpu.VMEM((9 * C, ts + 2 * hw), jnp.bfloat16)]

    cost1 = pl.CostEstimate(
        flops=2 * N * ms * (27 * C) * Cout,
        transcendentals=0,
        bytes_accessed=N * C * ms * 4 + 3 * 9 * C * Cout * 2
                       + N * Cout * ms * 2 + N * n_tiles * Cout * 2 * 4)

    y_cm, part_stats = pl.pallas_call(
        functools.partial(_conv_stats_kernel, **kw_args),
        out_shape=(jax.ShapeDtypeStruct((N, Cout, ms), jnp.bfloat16),
                   jax.ShapeDtypeStruct((N, n_tiles, Cout, 2), jnp.float32)),
        grid=grid,
        in_specs=[xb_spec, wk_spec],
        out_specs=[pl.BlockSpec((None, Cout, ts), lambda n, i: (n, 0, i)),
                   pl.BlockSpec((None, None, Cout, 2), lambda n, i: (n, i, 0, 0))],
        scratch_shapes=scratch,
        compiler_params=pltpu.CompilerParams(
            dimension_semantics=("parallel", "arbitrary")),
        cost_estimate=cost1,
    )(xb, wk)

    M = N * ms
    stats = jnp.sum(part_stats, axis=(0, 1))
    mean = stats[:, 0] / M
    var = jnp.maximum(stats[:, 1] / M - mean * mean, 0.0)
    scale = gamma / jnp.sqrt(var + _BN_EPS)
    shift = beta - mean * scale
    scale_c = scale.reshape(Cout, 1).astype(jnp.float32)
    shift_c = shift.reshape(Cout, 1).astype(jnp.float32)

    cost2 = pl.CostEstimate(
        flops=4 * N * ms * Cout,
        transcendentals=0,
        bytes_accessed=N * ms * Cout * (2 + 4) + 2 * Cout * 4)

    out_cm = pl.pallas_call(
        _bn_act_kernel,
        out_shape=jax.ShapeDtypeStruct((N, Cout, ms), jnp.float32),
        grid=grid,
        in_specs=[pl.BlockSpec((None, Cout, ts), lambda n, i: (n, 0, i)),
                  pl.BlockSpec((Cout, 1), lambda n, i: (0, 0)),
                  pl.BlockSpec((Cout, 1), lambda n, i: (0, 0))],
        out_specs=pl.BlockSpec((None, Cout, ts), lambda n, i: (n, 0, i)),
        compiler_params=pltpu.CompilerParams(
            dimension_semantics=("parallel", "parallel")),
        cost_estimate=cost2,
    )(y_cm, scale_c, shift_c)

    return out_cm.reshape(N, Cout, D, H, W)


# speedup vs baseline: 1.9357x; 1.0316x over previous
import functools

import jax
import jax.numpy as jnp
from jax.experimental import pallas as pl
from jax.experimental.pallas import tpu as pltpu

_NEG_SLOPE = 0.01
_BN_EPS = 1e-5
_D_TILE = 32
_LEAD = 128


def _conv_tile(xb_ref, wk_ref, wbuf_ref, col_ref, *, c_in, hw, n_tiles):
    mt = pl.program_id(1)
    ts = _D_TILE * hw
    tspan = ts + 2 * hw
    base = mt * ts

    lo = _LEAD
    hi = _LEAD + hw + ts
    wbuf_ref[:, 0:_LEAD] = jnp.zeros((c_in, _LEAD), jnp.float32)
    wbuf_ref[:, hi + hw:hi + hw + _LEAD] = jnp.zeros((c_in, _LEAD), jnp.float32)

    @pl.when(mt == 0)
    def _():
        wbuf_ref[:, lo:lo + hw] = jnp.zeros((c_in, hw), jnp.float32)

    @pl.when(mt > 0)
    def _():
        wbuf_ref[:, lo:lo + hw] = xb_ref[:, pl.ds(base - hw, hw)]

    wbuf_ref[:, lo + hw:lo + hw + ts] = xb_ref[:, pl.ds(base, ts)]

    @pl.when(mt == n_tiles - 1)
    def _():
        wbuf_ref[:, hi:hi + hw] = jnp.zeros((c_in, hw), jnp.float32)

    @pl.when(mt < n_tiles - 1)
    def _():
        wbuf_ref[:, hi:hi + hw] = xb_ref[:, pl.ds(base + ts, hw)]

    vw = wbuf_ref[...]
    wi = jax.lax.broadcasted_iota(jnp.int32, (c_in, tspan), 1)
    w_pos = wi % 32
    h_pos = (wi // 32) % 32
    for b in range(3):
        for cw in range(3):
            t = b * 3 + cw
            s = _LEAD + (b - 1) * 32 + (cw - 1)
            v = vw[:, s:s + tspan]
            bad = None
            if b == 0:
                bad = h_pos == 0
            elif b == 2:
                bad = h_pos == 31
            if cw == 0:
                bad = (w_pos == 0) if bad is None else bad | (w_pos == 0)
            elif cw == 2:
                bad = (w_pos == 31) if bad is None else bad | (w_pos == 31)
            if bad is not None:
                v = jnp.where(bad, 0.0, v)
            col_ref[t * c_in:(t + 1) * c_in, :] = v.astype(jnp.bfloat16)

    acc = None
    for a in range(3):
        xs = col_ref[:, a * hw:a * hw + ts]
        p = jax.lax.dot_general(
            xs, wk_ref[a],
            dimension_numbers=(((0,), (0,)), ((), ())),
            preferred_element_type=jnp.float32)
        acc = p if acc is None else acc + p
    return acc.T


def _conv_stats_kernel(xb_ref, wk_ref, y_ref, stats_ref, wbuf_ref, col_ref,
                       **kw):
    yt = _conv_tile(xb_ref, wk_ref, wbuf_ref, col_ref, **kw)
    y_ref[...] = yt.astype(y_ref.dtype)
    stats_ref[:, 0:1] = jnp.sum(yt, axis=-1, keepdims=True)
    stats_ref[:, 1:2] = jnp.sum(yt * yt, axis=-1, keepdims=True)


def _bn_act_kernel(y_ref, scale_ref, shift_ref, o_ref):
    y = y_ref[...].astype(jnp.float32) * scale_ref[...] + shift_ref[...]
    o_ref[...] = jnp.where(y >= 0.0, y, _NEG_SLOPE * y)


def kernel(w, gamma, beta, seqs, seqL):
    del seqL
    N, C, D, H, W = seqs.shape
    Cout = w.shape[0]
    hw = H * W
    ms = D * hw
    n_tiles = D // _D_TILE
    ts = _D_TILE * hw
    wspan = 2 * _LEAD + 2 * hw + ts

    xb = seqs.reshape(N, C, ms)
    wk = w.transpose(2, 3, 4, 1, 0).reshape(3, 9 * C, Cout).astype(jnp.bfloat16)

    kw_args = dict(c_in=C, hw=hw, n_tiles=n_tiles)
    grid = (N, n_tiles)
    xb_spec = pl.BlockSpec((None, C, ms), lambda n, i: (n, 0, 0))
    wk_spec = pl.BlockSpec((3, 9 * C, Cout), lambda n, i: (0, 0, 0))
    scratch = [pltpu.VMEM((C, wspan), jnp.float32),
               pltpu.VMEM((9 * C, ts + 2 * hw), jnp.bfloat16)]

    cost1 = pl.CostEstimate(
        flops=2 * N * ms * (27 * C) * Cout,
        transcendentals=0,
        bytes_accessed=N * C * ms * 4 + 3 * 9 * C * Cout * 2
                       + N * Cout * ms * 2 + N * n_tiles * Cout * 2 * 4)

    y_cm, part_stats = pl.pallas_call(
        functools.partial(_conv_stats_kernel, **kw_args),
        out_shape=(jax.ShapeDtypeStruct((N, Cout, ms), jnp.bfloat16),
                   jax.ShapeDtypeStruct((N, n_tiles, Cout, 2), jnp.float32)),
        grid=grid,
        in_specs=[xb_spec, wk_spec],
        out_specs=[pl.BlockSpec((None, Cout, ts), lambda n, i: (n, 0, i)),
                   pl.BlockSpec((None, None, Cout, 2), lambda n, i: (n, i, 0, 0))],
        scratch_shapes=scratch,
        compiler_params=pltpu.CompilerParams(
            dimension_semantics=("parallel", "arbitrary")),
        cost_estimate=cost1,
    )(xb, wk)

    M = N * ms
    stats = jnp.sum(part_stats, axis=(0, 1))
    mean = stats[:, 0] / M
    var = jnp.maximum(stats[:, 1] / M - mean * mean, 0.0)
    scale = gamma / jnp.sqrt(var + _BN_EPS)
    shift = beta - mean * scale
    scale_c = scale.reshape(Cout, 1).astype(jnp.float32)
    shift_c = shift.reshape(Cout, 1).astype(jnp.float32)

    cost2 = pl.CostEstimate(
        flops=4 * N * ms * Cout,
        transcendentals=0,
        bytes_accessed=N * ms * Cout * (2 + 4) + 2 * Cout * 4)

    out_cm = pl.pallas_call(
        _bn_act_kernel,
        out_shape=jax.ShapeDtypeStruct((N, Cout, ms), jnp.float32),
        grid=grid,
        in_specs=[pl.BlockSpec((None, Cout, ts), lambda n, i: (n, 0, i)),
                  pl.BlockSpec((Cout, 1), lambda n, i: (0, 0)),
                  pl.BlockSpec((Cout, 1), lambda n, i: (0, 0))],
        out_specs=pl.BlockSpec((None, Cout, ts), lambda n, i: (n, 0, i)),
        compiler_params=pltpu.CompilerParams(
            dimension_semantics=("parallel", "parallel")),
        cost_estimate=cost2,
    )(y_cm, scale_c, shift_c)

    return out_cm.reshape(N, Cout, D, H, W)
